# baseline (speedup 1.0000x reference)
import functools

import jax
import jax.numpy as jnp
from jax import lax
from jax.experimental import pallas as pl
from jax.experimental.pallas import tpu as pltpu
from jax.experimental.pallas import tpu_sc as plsc

_NC = 2
_NS = 16
_NW = _NC * _NS

_CHUNK = 2048
_SEG = 20000

_HALF = 0x8000
_LO = 0xFFFF
_HI = -65536


def _sc_gather_packed(idx, intercepts, slopes):
  n = idx.shape[0]
  s = intercepts.shape[0]
  per_w = n // _NW
  chunks = per_w // _CHUNK
  n_pieces = s // _SEG
  stage_iters = (n_pieces + _NS - 1) // _NS
  mesh = plsc.VectorSubcoreMesh(core_axis_name="c", subcore_axis_name="s")

  @functools.partial(
      pl.kernel,
      mesh=mesh,
      out_type=jax.ShapeDtypeStruct((n,), jnp.int32),
      scratch_types=[
          pltpu.VMEM_SHARED((s,), jnp.int32),
          pltpu.VMEM((_SEG,), jnp.float32),
          pltpu.VMEM((_SEG,), jnp.int32),
          pltpu.VMEM((_CHUNK,), jnp.int32),
          pltpu.VMEM((_CHUNK,), jnp.int32),
          pltpu.SemaphoreType.DMA,
      ],
  )
  def run(idx_hbm, ri_hbm, rs_hbm, out_hbm,
          s_tab, stage_f, stage_p, idx_v, p_v, sem):
    cid = lax.axis_index("c")
    sid = lax.axis_index("s")

    def stage_body(k, carry):
      piece = k * _NS + sid
      live = piece < n_pieces
      off = jnp.where(live, piece, 0) * _SEG

      @pl.when(live)
      def _():
        pltpu.sync_copy(ri_hbm.at[pl.ds(off, _SEG)], stage_f)

        def pack_lo(j, c2):
          sl = pl.ds(j * 16, 16)
          a = jax.lax.bitcast_convert_type(stage_f[sl], jnp.int32)
          stage_p[sl] = ((a + _HALF) >> 16) & _LO
          return c2

        lax.fori_loop(0, _SEG // 16, pack_lo, 0, unroll=8)
        pltpu.sync_copy(rs_hbm.at[pl.ds(off, _SEG)], stage_f)

        def pack_hi(j, c2):
          sl = pl.ds(j * 16, 16)
          b = jax.lax.bitcast_convert_type(stage_f[sl], jnp.int32)
          stage_p[sl] = stage_p[sl] | ((b + _HALF) & _HI)
          return c2

        lax.fori_loop(0, _SEG // 16, pack_hi, 0, unroll=8)
        pltpu.sync_copy(stage_p, s_tab.at[pl.ds(off, _SEG)])

      return carry

    lax.fori_loop(0, stage_iters, stage_body, 0)
    plsc.subcore_barrier()

    wid = sid * _NC + cid
    base0 = wid * per_w

    def chunk_body(ci, carry):
      base = base0 + ci * _CHUNK
      pltpu.sync_copy(idx_hbm.at[pl.ds(base, _CHUNK)], idx_v)
      pltpu.async_copy(s_tab.at[idx_v], p_v, sem).wait()
      pltpu.sync_copy(p_v, out_hbm.at[pl.ds(base, _CHUNK)])
      return carry

    lax.fori_loop(0, chunks, chunk_body, 0)

  return run(idx, intercepts, slopes)


_COLS = 32768


def _tc_matvec(xT, w):
  p, n = xT.shape

  def body(w_ref, x_ref, o_ref):
    o_ref[...] = jax.lax.dot_general(
        w_ref[...], x_ref[...], (((1,), (0,)), ((), ())),
        preferred_element_type=jnp.float32)

  return pl.pallas_call(
      body,
      grid=(n // _COLS,),
      in_specs=[
          pl.BlockSpec((1, p), lambda i: (0, 0)),
          pl.BlockSpec((p, _COLS), lambda i: (0, i)),
      ],
      out_specs=pl.BlockSpec((1, _COLS), lambda i: (0, i)),
      out_shape=jax.ShapeDtypeStruct((1, n), jnp.float32),
  )(w.reshape(1, p), xT)


_CCOLS = 131072


def _tc_combine(fe, pk, x):
  _, n = fe.shape

  def body(a_ref, p_ref, x_ref, o_ref):
    v = p_ref[...]
    ri = jax.lax.bitcast_convert_type(v << 16, jnp.float32)
    rs = jax.lax.bitcast_convert_type(v & _HI, jnp.float32)
    o_ref[...] = a_ref[...] + ri + x_ref[...] * rs

  return pl.pallas_call(
      body,
      grid=(n // _CCOLS,),
      in_specs=[
          pl.BlockSpec((1, _CCOLS), lambda i: (0, i)),
          pl.BlockSpec((1, _CCOLS), lambda i: (0, i)),
          pl.BlockSpec((1, _CCOLS), lambda i: (0, i)),
      ],
      out_specs=pl.BlockSpec((1, _CCOLS), lambda i: (0, i)),
      out_shape=jax.ShapeDtypeStruct((1, n), jnp.float32),
  )(fe, pk, x)


def kernel(X_fixed, subject_indices, X_random_slope, fixed_effects,
           random_intercepts, random_slopes):
  n, _, p = X_fixed.shape
  xT = jnp.transpose(X_fixed, (2, 1, 0)).reshape(p, n)
  idx = subject_indices.reshape(n)
  pk = _sc_gather_packed(idx, random_intercepts, random_slopes)
  fe = _tc_matvec(xT, fixed_effects)
  out = _tc_combine(fe, pk.reshape(1, n), X_random_slope.reshape(1, n))
  return out.reshape(n, 1)


# speedup vs baseline: 1.0987x; 1.0987x over previous
import functools

import jax
import jax.numpy as jnp
from jax import lax
from jax.experimental import pallas as pl
from jax.experimental.pallas import tpu as pltpu
from jax.experimental.pallas import tpu_sc as plsc

_NC = 2
_NS = 16
_NW = _NC * _NS

_CHUNK = 2048
_SEG = 20000

_HALF = 0x8000
_LO = 0xFFFF
_HI = -65536


def _sc_gather_packed(idx, packed):
  n = idx.shape[0]
  s = packed.shape[0]
  per_w = n // _NW
  chunks = per_w // _CHUNK
  n_pieces = s // _SEG
  stage_iters = (n_pieces + _NS - 1) // _NS
  mesh = plsc.VectorSubcoreMesh(core_axis_name="c", subcore_axis_name="s")

  @functools.partial(
      pl.kernel,
      mesh=mesh,
      out_type=jax.ShapeDtypeStruct((n,), jnp.int32),
      scratch_types=[
          pltpu.VMEM_SHARED((s,), jnp.int32),
          pltpu.VMEM((_SEG,), jnp.int32),
          pltpu.VMEM((_CHUNK,), jnp.int32),
          pltpu.VMEM((_CHUNK,), jnp.int32),
          pltpu.SemaphoreType.DMA,
      ],
  )
  def run(idx_hbm, tab_hbm, out_hbm, s_tab, stage_v, idx_v, p_v, sem):
    cid = lax.axis_index("c")
    sid = lax.axis_index("s")

    def stage_body(k, carry):
      piece = k * _NS + sid

      @pl.when(piece < n_pieces)
      def _():
        off = piece * _SEG
        pltpu.sync_copy(tab_hbm.at[pl.ds(off, _SEG)], stage_v)
        pltpu.sync_copy(stage_v, s_tab.at[pl.ds(off, _SEG)])

      return carry

    lax.fori_loop(0, stage_iters, stage_body, 0)
    plsc.subcore_barrier()

    wid = sid * _NC + cid
    base0 = wid * per_w

    def chunk_body(ci, carry):
      base = base0 + ci * _CHUNK
      pltpu.sync_copy(idx_hbm.at[pl.ds(base, _CHUNK)], idx_v)
      pltpu.async_copy(s_tab.at[idx_v], p_v, sem).wait()
      pltpu.sync_copy(p_v, out_hbm.at[pl.ds(base, _CHUNK)])
      return carry

    lax.fori_loop(0, chunks, chunk_body, 0)

  return run(idx, packed)


_COLS = 32768


def _tc_matvec(xT, w):
  p, n = xT.shape

  def body(w_ref, x_ref, o_ref):
    o_ref[...] = jax.lax.dot_general(
        w_ref[...], x_ref[...], (((1,), (0,)), ((), ())),
        preferred_element_type=jnp.float32)

  return pl.pallas_call(
      body,
      grid=(n // _COLS,),
      in_specs=[
          pl.BlockSpec((1, p), lambda i: (0, 0)),
          pl.BlockSpec((p, _COLS), lambda i: (0, i)),
      ],
      out_specs=pl.BlockSpec((1, _COLS), lambda i: (0, i)),
      out_shape=jax.ShapeDtypeStruct((1, n), jnp.float32),
  )(w.reshape(1, p), xT)


_CCOLS = 131072


def _tc_combine(fe, pk, x):
  _, n = fe.shape

  def body(a_ref, p_ref, x_ref, o_ref):
    v = p_ref[...]
    ri = jax.lax.bitcast_convert_type(v << 16, jnp.float32)
    rs = jax.lax.bitcast_convert_type(v & _HI, jnp.float32)
    o_ref[...] = a_ref[...] + ri + x_ref[...] * rs

  return pl.pallas_call(
      body,
      grid=(n // _CCOLS,),
      in_specs=[
          pl.BlockSpec((1, _CCOLS), lambda i: (0, i)),
          pl.BlockSpec((1, _CCOLS), lambda i: (0, i)),
          pl.BlockSpec((1, _CCOLS), lambda i: (0, i)),
      ],
      out_specs=pl.BlockSpec((1, _CCOLS), lambda i: (0, i)),
      out_shape=jax.ShapeDtypeStruct((1, n), jnp.float32),
  )(fe, pk, x)


def kernel(X_fixed, subject_indices, X_random_slope, fixed_effects,
           random_intercepts, random_slopes):
  n, _, p = X_fixed.shape
  xT = jnp.transpose(X_fixed, (2, 1, 0)).reshape(p, n)
  idx = subject_indices.reshape(n)
  ri_u = jax.lax.bitcast_convert_type(
      random_intercepts.astype(jnp.bfloat16), jnp.uint16).astype(jnp.uint32)
  rs_u = jax.lax.bitcast_convert_type(
      random_slopes.astype(jnp.bfloat16), jnp.uint16).astype(jnp.uint32)
  packed = jax.lax.bitcast_convert_type(ri_u | (rs_u << 16), jnp.int32)
  pk = _sc_gather_packed(idx, packed)
  fe = _tc_matvec(xT, fixed_effects)
  out = _tc_combine(fe, pk.reshape(1, n), X_random_slope.reshape(1, n))
  return out.reshape(n, 1)


# speedup vs baseline: 1.1008x; 1.0018x over previous
import functools

import jax
import jax.numpy as jnp
from jax import lax
from jax.experimental import pallas as pl
from jax.experimental.pallas import tpu as pltpu
from jax.experimental.pallas import tpu_sc as plsc

_NC = 2
_NS = 16
_NW = _NC * _NS

_CHUNK = 2048
_SEG = 20000

_HALF = 0x8000
_LO = 0xFFFF
_HI = -65536

_DIAG = ""


def _sc_gather_packed(idx, packed):
  n = idx.shape[0]
  s = packed.shape[0]
  per_w = n // _NW
  chunks = per_w // _CHUNK
  n_pieces = s // _SEG
  stage_iters = (n_pieces + _NS - 1) // _NS
  mesh = plsc.VectorSubcoreMesh(core_axis_name="c", subcore_axis_name="s")

  @functools.partial(
      pl.kernel,
      mesh=mesh,
      out_type=jax.ShapeDtypeStruct((n,), jnp.int32),
      scratch_types=[
          pltpu.VMEM_SHARED((s,), jnp.int32),
          pltpu.VMEM((_SEG,), jnp.int32),
          pltpu.VMEM((_CHUNK,), jnp.int32),
          pltpu.VMEM((_CHUNK,), jnp.int32),
          pltpu.SemaphoreType.DMA,
      ],
  )
  def run(idx_hbm, tab_hbm, out_hbm, s_tab, stage_v, idx_v, p_v, sem):
    cid = lax.axis_index("c")
    sid = lax.axis_index("s")

    def stage_body(k, carry):
      piece = k * _NS + sid

      @pl.when(piece < n_pieces)
      def _():
        off = piece * _SEG
        pltpu.sync_copy(tab_hbm.at[pl.ds(off, _SEG)], stage_v)
        pltpu.sync_copy(stage_v, s_tab.at[pl.ds(off, _SEG)])

      return carry

    lax.fori_loop(0, stage_iters, stage_body, 0)
    plsc.subcore_barrier()

    wid = sid * _NC + cid
    base0 = wid * per_w

    def chunk_body(ci, carry):
      base = base0 + ci * _CHUNK
      pltpu.sync_copy(idx_hbm.at[pl.ds(base, _CHUNK)], idx_v)
      pltpu.async_copy(s_tab.at[idx_v], p_v, sem).wait()
      pltpu.sync_copy(p_v, out_hbm.at[pl.ds(base, _CHUNK)])
      return carry

    lax.fori_loop(0, chunks, chunk_body, 0)

  return run(idx, packed)


_COLS = 32768


def _tc_matvec(xT, w):
  p, n = xT.shape

  def body(w_ref, x_ref, o_ref):
    o_ref[...] = jax.lax.dot_general(
        w_ref[...], x_ref[...], (((1,), (0,)), ((), ())),
        preferred_element_type=jnp.float32)

  return pl.pallas_call(
      body,
      grid=(n // _COLS,),
      in_specs=[
          pl.BlockSpec((1, p), lambda i: (0, 0)),
          pl.BlockSpec((p, _COLS), lambda i: (0, i)),
      ],
      out_specs=pl.BlockSpec((1, _COLS), lambda i: (0, i)),
      out_shape=jax.ShapeDtypeStruct((1, n), jnp.float32),
  )(w.reshape(1, p), xT)


_CCOLS = 131072


def _tc_combine(fe, pk, x):
  _, n = fe.shape

  def body(a_ref, p_ref, x_ref, o_ref):
    v = p_ref[...]
    ri = jax.lax.bitcast_convert_type(v << 16, jnp.float32)
    rs = jax.lax.bitcast_convert_type(v & _HI, jnp.float32)
    o_ref[...] = a_ref[...] + ri + x_ref[...] * rs

  return pl.pallas_call(
      body,
      grid=(n // _CCOLS,),
      in_specs=[
          pl.BlockSpec((1, _CCOLS), lambda i: (0, i)),
          pl.BlockSpec((1, _CCOLS), lambda i: (0, i)),
          pl.BlockSpec((1, _CCOLS), lambda i: (0, i)),
      ],
      out_specs=pl.BlockSpec((1, _CCOLS), lambda i: (0, i)),
      out_shape=jax.ShapeDtypeStruct((1, n), jnp.float32),
  )(fe, pk, x)


def kernel(X_fixed, subject_indices, X_random_slope, fixed_effects,
           random_intercepts, random_slopes):
  n, _, p = X_fixed.shape
  xT = jnp.transpose(X_fixed, (2, 1, 0)).reshape(p, n)
  idx = subject_indices.reshape(n)
  ri_u = jax.lax.bitcast_convert_type(
      random_intercepts.astype(jnp.bfloat16), jnp.uint16).astype(jnp.uint32)
  rs_u = jax.lax.bitcast_convert_type(
      random_slopes.astype(jnp.bfloat16), jnp.uint16).astype(jnp.uint32)
  packed = jax.lax.bitcast_convert_type(ri_u | (rs_u << 16), jnp.int32)
  pk = _sc_gather_packed(idx, packed)
  fe = _tc_matvec(xT, fixed_effects)
  if _DIAG == "matvec":
    return fe.reshape(n, 1)
  if _DIAG == "sc":
    return jax.lax.bitcast_convert_type(pk, jnp.float32).reshape(n, 1)
  out = _tc_combine(fe, pk.reshape(1, n), X_random_slope.reshape(1, n))
  return out.reshape(n, 1)


# speedup vs baseline: 1.1302x; 1.0267x over previous
import functools

import jax
import jax.numpy as jnp
from jax import lax
from jax.experimental import pallas as pl
from jax.experimental.pallas import tpu as pltpu
from jax.experimental.pallas import tpu_sc as plsc

_NC = 2
_NS = 16
_NW = _NC * _NS

_CHUNK = 2048
_SEG = 20000

_HALF = 0x8000
_LO = 0xFFFF
_HI = -65536

_DIAG = ""


def _sc_gather_packed(idx, packed):
  n = idx.shape[0]
  s = packed.shape[0]
  per_w = n // _NW
  chunks = per_w // _CHUNK
  n_pieces = s // _SEG
  stage_iters = (n_pieces + _NS - 1) // _NS
  mesh = plsc.VectorSubcoreMesh(core_axis_name="c", subcore_axis_name="s")

  @functools.partial(
      pl.kernel,
      mesh=mesh,
      out_type=jax.ShapeDtypeStruct((n,), jnp.int32),
      scratch_types=[
          pltpu.VMEM_SHARED((s,), jnp.int32),
          pltpu.VMEM((_SEG,), jnp.int32),
          pltpu.VMEM((_CHUNK,), jnp.int32),
          pltpu.VMEM((_CHUNK,), jnp.int32),
          pltpu.SemaphoreType.DMA,
      ],
  )
  def run(idx_hbm, tab_hbm, out_hbm, s_tab, stage_v, idx_v, p_v, sem):
    cid = lax.axis_index("c")
    sid = lax.axis_index("s")

    def stage_body(k, carry):
      piece = k * _NS + sid

      @pl.when(piece < n_pieces)
      def _():
        off = piece * _SEG
        pltpu.sync_copy(tab_hbm.at[pl.ds(off, _SEG)], stage_v)
        pltpu.sync_copy(stage_v, s_tab.at[pl.ds(off, _SEG)])

      return carry

    lax.fori_loop(0, stage_iters, stage_body, 0)
    plsc.subcore_barrier()

    wid = sid * _NC + cid
    base0 = wid * per_w

    def chunk_body(ci, carry):
      base = base0 + ci * _CHUNK
      pltpu.sync_copy(idx_hbm.at[pl.ds(base, _CHUNK)], idx_v)
      pltpu.async_copy(s_tab.at[idx_v], p_v, sem).wait()
      pltpu.sync_copy(p_v, out_hbm.at[pl.ds(base, _CHUNK)])
      return carry

    lax.fori_loop(0, chunks, chunk_body, 0)

  return run(idx, packed)


_COLS = 32768


def _tc_matvec(xT, w):
  p, n = xT.shape

  def body(w_ref, x_ref, o_ref):
    o_ref[...] = jax.lax.dot_general(
        w_ref[...], x_ref[...], (((1,), (0,)), ((), ())),
        preferred_element_type=jnp.float32)

  return pl.pallas_call(
      body,
      grid=(n // _COLS,),
      in_specs=[
          pl.BlockSpec((1, p), lambda i: (0, 0)),
          pl.BlockSpec((p, _COLS), lambda i: (0, i)),
      ],
      out_specs=pl.BlockSpec((1, _COLS), lambda i: (0, i)),
      out_shape=jax.ShapeDtypeStruct((1, n), jnp.float32),
  )(w.reshape(1, p), xT)


_CROWS = 1024


def _tc_combine(fe, pk, x):
  r, c = fe.shape

  def body(a_ref, p_ref, x_ref, o_ref):
    v = p_ref[...]
    ri = jax.lax.bitcast_convert_type(v << 16, jnp.float32)
    rs = jax.lax.bitcast_convert_type(v & _HI, jnp.float32)
    o_ref[...] = a_ref[...] + ri + x_ref[...] * rs

  return pl.pallas_call(
      body,
      grid=(r // _CROWS,),
      in_specs=[
          pl.BlockSpec((_CROWS, c), lambda i: (i, 0)),
          pl.BlockSpec((_CROWS, c), lambda i: (i, 0)),
          pl.BlockSpec((_CROWS, c), lambda i: (i, 0)),
      ],
      out_specs=pl.BlockSpec((_CROWS, c), lambda i: (i, 0)),
      out_shape=jax.ShapeDtypeStruct((r, c), jnp.float32),
  )(fe, pk, x)


def kernel(X_fixed, subject_indices, X_random_slope, fixed_effects,
           random_intercepts, random_slopes):
  n, _, p = X_fixed.shape
  xT = jnp.transpose(X_fixed, (2, 1, 0)).reshape(p, n)
  idx = subject_indices.reshape(n)
  ri_u = jax.lax.bitcast_convert_type(
      random_intercepts.astype(jnp.bfloat16), jnp.uint16).astype(jnp.uint32)
  rs_u = jax.lax.bitcast_convert_type(
      random_slopes.astype(jnp.bfloat16), jnp.uint16).astype(jnp.uint32)
  packed = jax.lax.bitcast_convert_type(ri_u | (rs_u << 16), jnp.int32)
  pk = _sc_gather_packed(idx, packed)
  fe = _tc_matvec(xT, fixed_effects)
  if _DIAG == "matvec":
    return fe.reshape(n, 1)
  if _DIAG == "sc":
    return jax.lax.bitcast_convert_type(pk, jnp.float32).reshape(n, 1)
  out = _tc_combine(fe.reshape(n // 128, 128), pk.reshape(n // 128, 128),
                    X_random_slope.reshape(n // 128, 128))
  return out.reshape(n, 1)


# speedup vs baseline: 1.1353x; 1.0045x over previous
import functools

import jax
import jax.numpy as jnp
from jax import lax
from jax.experimental import pallas as pl
from jax.experimental.pallas import tpu as pltpu
from jax.experimental.pallas import tpu_sc as plsc

_NC = 2
_NS = 16
_NW = _NC * _NS

_CHUNK = 2048
_SEG = 20000

_HALF = 0x8000
_LO = 0xFFFF
_HI = -65536

_DIAG = ""


def _sc_gather_packed(idx, packed):
  n = idx.shape[0]
  s = packed.shape[0]
  per_w = n // _NW
  chunks = per_w // _CHUNK
  n_pieces = s // _SEG
  stage_iters = (n_pieces + _NS - 1) // _NS
  mesh = plsc.VectorSubcoreMesh(core_axis_name="c", subcore_axis_name="s")

  @functools.partial(
      pl.kernel,
      mesh=mesh,
      out_type=jax.ShapeDtypeStruct((n,), jnp.int32),
      scratch_types=[
          pltpu.VMEM_SHARED((s,), jnp.int32),
          pltpu.VMEM((_SEG,), jnp.int32),
          pltpu.VMEM((_CHUNK,), jnp.int32),
          pltpu.VMEM((_CHUNK,), jnp.int32),
          pltpu.SemaphoreType.DMA,
      ],
  )
  def run(idx_hbm, tab_hbm, out_hbm, s_tab, stage_v, idx_v, p_v, sem):
    cid = lax.axis_index("c")
    sid = lax.axis_index("s")

    def stage_body(k, carry):
      piece = k * _NS + sid

      @pl.when(piece < n_pieces)
      def _():
        off = piece * _SEG
        pltpu.sync_copy(tab_hbm.at[pl.ds(off, _SEG)], stage_v)
        pltpu.sync_copy(stage_v, s_tab.at[pl.ds(off, _SEG)])

      return carry

    lax.fori_loop(0, stage_iters, stage_body, 0)
    plsc.subcore_barrier()

    wid = sid * _NC + cid
    base0 = wid * per_w

    def chunk_body(ci, carry):
      base = base0 + ci * _CHUNK
      pltpu.sync_copy(idx_hbm.at[pl.ds(base, _CHUNK)], idx_v)
      pltpu.async_copy(s_tab.at[idx_v], p_v, sem).wait()
      pltpu.sync_copy(p_v, out_hbm.at[pl.ds(base, _CHUNK)])
      return carry

    lax.fori_loop(0, chunks, chunk_body, 0)

  return run(idx, packed)


_COLS = 65536


def _tc_matvec(xT, w):
  p, n = xT.shape

  def body(w_ref, x_ref, o_ref):
    o_ref[...] = jax.lax.dot_general(
        w_ref[...], x_ref[...], (((1,), (0,)), ((), ())),
        preferred_element_type=jnp.float32)

  return pl.pallas_call(
      body,
      grid=(n // _COLS,),
      in_specs=[
          pl.BlockSpec((1, p), lambda i: (0, 0)),
          pl.BlockSpec((p, _COLS), lambda i: (0, i)),
      ],
      out_specs=pl.BlockSpec((1, _COLS), lambda i: (0, i)),
      out_shape=jax.ShapeDtypeStruct((1, n), jnp.float32),
  )(w.reshape(1, p), xT)


_CROWS = 2048


def _tc_combine(fe, pk, x):
  r, c = fe.shape

  def body(a_ref, p_ref, x_ref, o_ref):
    v = p_ref[...]
    ri = jax.lax.bitcast_convert_type(v << 16, jnp.float32)
    rs = jax.lax.bitcast_convert_type(v & _HI, jnp.float32)
    o_ref[...] = a_ref[...] + ri + x_ref[...] * rs

  return pl.pallas_call(
      body,
      grid=(r // _CROWS,),
      in_specs=[
          pl.BlockSpec((_CROWS, c), lambda i: (i, 0)),
          pl.BlockSpec((_CROWS, c), lambda i: (i, 0)),
          pl.BlockSpec((_CROWS, c), lambda i: (i, 0)),
      ],
      out_specs=pl.BlockSpec((_CROWS, c), lambda i: (i, 0)),
      out_shape=jax.ShapeDtypeStruct((r, c), jnp.float32),
  )(fe, pk, x)


def kernel(X_fixed, subject_indices, X_random_slope, fixed_effects,
           random_intercepts, random_slopes):
  n, _, p = X_fixed.shape
  xT = jnp.transpose(X_fixed, (2, 1, 0)).reshape(p, n)
  idx = subject_indices.reshape(n)
  ri_u = jax.lax.bitcast_convert_type(
      random_intercepts.astype(jnp.bfloat16), jnp.uint16).astype(jnp.uint32)
  rs_u = jax.lax.bitcast_convert_type(
      random_slopes.astype(jnp.bfloat16), jnp.uint16).astype(jnp.uint32)
  packed = jax.lax.bitcast_convert_type(ri_u | (rs_u << 16), jnp.int32)
  pk = _sc_gather_packed(idx, packed)
  fe = _tc_matvec(xT, fixed_effects)
  if _DIAG == "matvec":
    return fe.reshape(n, 1)
  if _DIAG == "sc":
    return jax.lax.bitcast_convert_type(pk, jnp.float32).reshape(n, 1)
  out = _tc_combine(fe.reshape(n // 128, 128), pk.reshape(n // 128, 128),
                    X_random_slope.reshape(n // 128, 128))
  return out.reshape(n, 1)


# speedup vs baseline: 1.1604x; 1.0221x over previous
import functools

import jax
import jax.numpy as jnp
from jax import lax
from jax.experimental import pallas as pl
from jax.experimental.pallas import tpu as pltpu
from jax.experimental.pallas import tpu_sc as plsc

_NC = 2
_NS = 16
_NW = _NC * _NS

_CHUNK = 2048
_SEG = 20000

_HALF = 0x8000
_LO = 0xFFFF
_HI = -65536

_DIAG = ""


def _sc_gather_packed(idx, intercepts, slopes):
  n = idx.shape[0]
  s = intercepts.shape[0]
  per_w = n // _NW
  chunks = per_w // _CHUNK
  n_pieces = s // _SEG
  stage_iters = (n_pieces + _NS - 1) // _NS
  mesh = plsc.VectorSubcoreMesh(core_axis_name="c", subcore_axis_name="s")

  @functools.partial(
      pl.kernel,
      mesh=mesh,
      out_type=jax.ShapeDtypeStruct((n,), jnp.int32),
      scratch_types=[
          pltpu.VMEM_SHARED((s,), jnp.int32),
          pltpu.VMEM((_SEG,), jnp.float32),
          pltpu.VMEM((_SEG,), jnp.float32),
          pltpu.VMEM((_SEG,), jnp.int32),
          pltpu.VMEM((_CHUNK,), jnp.int32),
          pltpu.VMEM((_CHUNK,), jnp.int32),
          pltpu.SemaphoreType.DMA,
      ],
  )
  def run(idx_hbm, ri_hbm, rs_hbm, out_hbm,
          s_tab, stage_a, stage_b, stage_p, idx_v, p_v, sem):
    cid = lax.axis_index("c")
    sid = lax.axis_index("s")

    def stage_body(k, carry):
      piece = k * _NS + sid

      @pl.when(piece < n_pieces)
      def _():
        off = piece * _SEG
        pltpu.sync_copy(ri_hbm.at[pl.ds(off, _SEG)], stage_a)
        pltpu.sync_copy(rs_hbm.at[pl.ds(off, _SEG)], stage_b)

        def pack(j, c2):
          sl = pl.ds(j * 16, 16)
          a = jax.lax.bitcast_convert_type(stage_a[sl], jnp.int32)
          b = jax.lax.bitcast_convert_type(stage_b[sl], jnp.int32)
          stage_p[sl] = (((a + _HALF) >> 16) & _LO) | ((b + _HALF) & _HI)
          return c2

        lax.fori_loop(0, _SEG // 16, pack, 0, unroll=8)
        pltpu.sync_copy(stage_p, s_tab.at[pl.ds(off, _SEG)])

      return carry

    lax.fori_loop(0, stage_iters, stage_body, 0)
    plsc.subcore_barrier()

    wid = sid * _NC + cid
    base0 = wid * per_w

    def chunk_body(ci, carry):
      base = base0 + ci * _CHUNK
      pltpu.sync_copy(idx_hbm.at[pl.ds(base, _CHUNK)], idx_v)
      pltpu.async_copy(s_tab.at[idx_v], p_v, sem).wait()
      pltpu.sync_copy(p_v, out_hbm.at[pl.ds(base, _CHUNK)])
      return carry

    lax.fori_loop(0, chunks, chunk_body, 0)

  return run(idx, intercepts, slopes)


_COLS = 65536


def _tc_matvec(xT, w):
  p, n = xT.shape

  def body(w_ref, x_ref, o_ref):
    o_ref[...] = jax.lax.dot_general(
        w_ref[...], x_ref[...], (((1,), (0,)), ((), ())),
        preferred_element_type=jnp.float32)

  return pl.pallas_call(
      body,
      grid=(n // _COLS,),
      in_specs=[
          pl.BlockSpec((1, p), lambda i: (0, 0)),
          pl.BlockSpec((p, _COLS), lambda i: (0, i)),
      ],
      out_specs=pl.BlockSpec((1, _COLS), lambda i: (0, i)),
      out_shape=jax.ShapeDtypeStruct((1, n), jnp.float32),
  )(w.reshape(1, p), xT)


_CROWS = 2048


def _tc_combine(fe, pk, x):
  r, c = fe.shape

  def body(a_ref, p_ref, x_ref, o_ref):
    v = p_ref[...]
    ri = jax.lax.bitcast_convert_type(v << 16, jnp.float32)
    rs = jax.lax.bitcast_convert_type(v & _HI, jnp.float32)
    o_ref[...] = a_ref[...] + ri + x_ref[...] * rs

  return pl.pallas_call(
      body,
      grid=(r // _CROWS,),
      in_specs=[
          pl.BlockSpec((_CROWS, c), lambda i: (i, 0)),
          pl.BlockSpec((_CROWS, c), lambda i: (i, 0)),
          pl.BlockSpec((_CROWS, c), lambda i: (i, 0)),
      ],
      out_specs=pl.BlockSpec((_CROWS, c), lambda i: (i, 0)),
      out_shape=jax.ShapeDtypeStruct((r, c), jnp.float32),
  )(fe, pk, x)


def kernel(X_fixed, subject_indices, X_random_slope, fixed_effects,
           random_intercepts, random_slopes):
  n, _, p = X_fixed.shape
  xT = jnp.transpose(X_fixed, (2, 1, 0)).reshape(p, n)
  idx = subject_indices.reshape(n)
  pk = _sc_gather_packed(idx, random_intercepts, random_slopes)
  fe = _tc_matvec(xT, fixed_effects)
  if _DIAG == "matvec":
    return fe.reshape(n, 1)
  if _DIAG == "sc":
    return jax.lax.bitcast_convert_type(pk, jnp.float32).reshape(n, 1)
  out = _tc_combine(fe.reshape(n // 128, 128), pk.reshape(n // 128, 128),
                    X_random_slope.reshape(n // 128, 128))
  return out.reshape(n, 1)


# speedup vs baseline: 1.1644x; 1.0035x over previous
import functools

import jax
import jax.numpy as jnp
from jax import lax
from jax.experimental import pallas as pl
from jax.experimental.pallas import tpu as pltpu
from jax.experimental.pallas import tpu_sc as plsc

_NC = 2
_NS = 16
_NW = _NC * _NS

_CHUNK = 2048
_SEG = 20000

_HALF = 0x8000
_LO = 0xFFFF
_HI = -65536


def _sc_gather_packed(idx, intercepts, slopes):
  n = idx.shape[0]
  s = intercepts.shape[0]
  per_w = n // _NW
  chunks = per_w // _CHUNK
  n_pieces = s // _SEG
  stage_iters = (n_pieces + _NS - 1) // _NS
  mesh = plsc.VectorSubcoreMesh(core_axis_name="c", subcore_axis_name="s")

  @functools.partial(
      pl.kernel,
      mesh=mesh,
      out_type=jax.ShapeDtypeStruct((n,), jnp.int32),
      scratch_types=[
          pltpu.VMEM_SHARED((s,), jnp.int32),
          pltpu.VMEM((_SEG,), jnp.float32),
          pltpu.VMEM((_SEG,), jnp.float32),
          pltpu.VMEM((_SEG,), jnp.int32),
          pltpu.VMEM((_CHUNK,), jnp.int32),
          pltpu.VMEM((_CHUNK,), jnp.int32),
          pltpu.SemaphoreType.DMA,
      ],
  )
  def run(idx_hbm, ri_hbm, rs_hbm, out_hbm,
          s_tab, stage_a, stage_b, stage_p, idx_v, p_v, sem):
    cid = lax.axis_index("c")
    sid = lax.axis_index("s")

    def stage_body(k, carry):
      piece = k * _NS + sid

      @pl.when(piece < n_pieces)
      def _():
        off = piece * _SEG
        pltpu.sync_copy(ri_hbm.at[pl.ds(off, _SEG)], stage_a)
        pltpu.sync_copy(rs_hbm.at[pl.ds(off, _SEG)], stage_b)

        def pack(j, c2):
          sl = pl.ds(j * 16, 16)
          a = jax.lax.bitcast_convert_type(stage_a[sl], jnp.int32)
          b = jax.lax.bitcast_convert_type(stage_b[sl], jnp.int32)
          stage_p[sl] = (((a + _HALF) >> 16) & _LO) | ((b + _HALF) & _HI)
          return c2

        lax.fori_loop(0, _SEG // 16, pack, 0, unroll=8)
        pltpu.sync_copy(stage_p, s_tab.at[pl.ds(off, _SEG)])

      return carry

    lax.fori_loop(0, stage_iters, stage_body, 0)
    plsc.subcore_barrier()

    wid = sid * _NC + cid
    base0 = wid * per_w

    def chunk_body(ci, carry):
      base = base0 + ci * _CHUNK
      pltpu.sync_copy(idx_hbm.at[pl.ds(base, _CHUNK)], idx_v)
      pltpu.async_copy(s_tab.at[idx_v], p_v, sem).wait()
      pltpu.sync_copy(p_v, out_hbm.at[pl.ds(base, _CHUNK)])
      return carry

    lax.fori_loop(0, chunks, chunk_body, 0)

  return run(idx, intercepts, slopes)


_COLS = 65536


def _tc_matvec(xT, w):
  p, n = xT.shape

  def body(w_ref, x_ref, o_ref):
    o_ref[...] = jax.lax.dot_general(
        w_ref[...], x_ref[...], (((1,), (0,)), ((), ())),
        preferred_element_type=jnp.float32)

  return pl.pallas_call(
      body,
      grid=(n // _COLS,),
      in_specs=[
          pl.BlockSpec((1, p), lambda i: (0, 0)),
          pl.BlockSpec((p, _COLS), lambda i: (0, i)),
      ],
      out_specs=pl.BlockSpec((1, _COLS), lambda i: (0, i)),
      out_shape=jax.ShapeDtypeStruct((1, n), jnp.float32),
  )(w.reshape(1, p), xT)


_CROWS = 2048


def _tc_combine(fe, pk, x):
  r, c = fe.shape

  def body(a_ref, p_ref, x_ref, o_ref):
    v = p_ref[...]
    ri = jax.lax.bitcast_convert_type(v << 16, jnp.float32)
    rs = jax.lax.bitcast_convert_type(v & _HI, jnp.float32)
    o_ref[...] = a_ref[...] + ri + x_ref[...] * rs

  return pl.pallas_call(
      body,
      grid=(r // _CROWS,),
      in_specs=[
          pl.BlockSpec((_CROWS, c), lambda i: (i, 0)),
          pl.BlockSpec((_CROWS, c), lambda i: (i, 0)),
          pl.BlockSpec((_CROWS, c), lambda i: (i, 0)),
      ],
      out_specs=pl.BlockSpec((_CROWS, c), lambda i: (i, 0)),
      out_shape=jax.ShapeDtypeStruct((r, c), jnp.float32),
  )(fe, pk, x)


def kernel(X_fixed, subject_indices, X_random_slope, fixed_effects,
           random_intercepts, random_slopes):
  n, _, p = X_fixed.shape
  xT = jnp.transpose(X_fixed, (2, 1, 0)).reshape(p, n)
  idx = subject_indices.reshape(n)
  pk = _sc_gather_packed(idx, random_intercepts, random_slopes)
  fe = _tc_matvec(xT, fixed_effects)
  out = _tc_combine(fe.reshape(n // 128, 128), pk.reshape(n // 128, 128),
                    X_random_slope.reshape(n // 128, 128))
  return out.reshape(n, 1)
